# baseline (speedup 1.0000x reference)
import functools

import jax
import jax.numpy as jnp
from jax import lax
from jax.experimental import pallas as pl
from jax.experimental.pallas import tpu as pltpu
from jax.experimental.pallas import tpu_sc as plsc

NUM_MONTHS = 12
EMBED = 128
BATCH = 16384

_NC = 2
_NS = 16
_NW = _NC * _NS
_BPW = BATCH // _NW
_IDX_MINOR = 128
_NCHUNK = _BPW // _IDX_MINOR


def _vreg_gather(v, idx):
  return lax.gather(
      v,
      idx[:, None],
      dimension_numbers=lax.GatherDimensionNumbers(
          offset_dims=(), collapsed_slice_dims=(0,), start_index_map=(0,)),
      slice_sizes=(1,),
      mode=lax.GatherScatterMode.PROMISE_IN_BOUNDS,
  )


def _make_kernel():
  mesh = plsc.VectorSubcoreMesh(core_axis_name="c", subcore_axis_name="s")

  @functools.partial(
      pl.kernel,
      mesh=mesh,
      out_type=jax.ShapeDtypeStruct((BATCH, EMBED), jnp.float32),
      scratch_types=[
          pltpu.VMEM((_BPW * 2,), jnp.int32),
          pltpu.VMEM((_NCHUNK, _IDX_MINOR), jnp.int32),
          pltpu.VMEM((_BPW, EMBED), jnp.float32),
          pltpu.SemaphoreType.DMA,
      ],
  )
  def k(ti_hbm, table_hbm, out_hbm, ti_v, idx_v, rows_v, sem):
    wid = lax.axis_index("s") * _NC + lax.axis_index("c")
    base = wid * _BPW

    pltpu.sync_copy(ti_hbm.at[pl.ds(base * 2, _BPW * 2)], ti_v)

    lane = lax.iota(jnp.int32, 16)
    even = lane * 2
    evenw = even - 16
    in_lo = lane < 8
    for i in range(_BPW // 16):
      a = ti_v[pl.ds(32 * i, 16)]
      b = ti_v[pl.ds(32 * i + 16, 16)]
      ga = _vreg_gather(a, jnp.where(in_lo, even, 0))
      gb = _vreg_gather(b, jnp.where(in_lo, 0, evenw))
      months = jnp.where(in_lo, ga, gb)
      idx_v[i // 8, pl.ds((i % 8) * 16, 16)] = months

    copies = []
    for j in range(_NCHUNK):
      copies.append(
          pltpu.async_copy(
              table_hbm.at[idx_v.at[j]],
              rows_v.at[pl.ds(j * _IDX_MINOR, _IDX_MINOR)],
              sem,
          )
      )
    for c in copies:
      c.wait()

    pltpu.sync_copy(rows_v, out_hbm.at[pl.ds(base, _BPW)])

  return k


_sc_lookup = jax.jit(_make_kernel())


def kernel(time_input, month_table):
  return _sc_lookup(time_input.astype(jnp.int32).reshape(-1), month_table)


# speedup vs baseline: 1.9411x; 1.9411x over previous
import functools

import jax
import jax.numpy as jnp
from jax import lax
from jax.experimental import pallas as pl
from jax.experimental.pallas import tpu as pltpu
from jax.experimental.pallas import tpu_sc as plsc

NUM_MONTHS = 12
EMBED = 128
BATCH = 16384

_NC = 2
_NS = 16
_NW = _NC * _NS
_BPW = BATCH // _NW
_CHUNK = 128
_NCHUNK = _BPW // _CHUNK
_ROWS_PER_STEP = 8


def _make_kernel():
  mesh = plsc.VectorSubcoreMesh(core_axis_name="c", subcore_axis_name="s")

  @functools.partial(
      pl.kernel,
      mesh=mesh,
      out_type=jax.ShapeDtypeStruct((BATCH * EMBED,), jnp.float32),
      scratch_types=[
          pltpu.VMEM((NUM_MONTHS * EMBED,), jnp.float32),
          pltpu.VMEM((_BPW * 2,), jnp.int32),
          pltpu.VMEM((_BPW * EMBED,), jnp.float32),
          pltpu.SemaphoreType.DMA,
          pltpu.SemaphoreType.DMA,
      ],
  )
  def k(ti_hbm, table_hbm, out_hbm, table_v, ti_v, rows_v, in_sem, out_sem):
    wid = lax.axis_index("s") * _NC + lax.axis_index("c")
    base = wid * _BPW

    load_table = pltpu.async_copy(table_hbm, table_v, in_sem)
    pltpu.sync_copy(ti_hbm.at[pl.ds(base * 2, _BPW * 2)], ti_v)
    load_table.wait()

    out_copies = []
    for c in range(_NCHUNK):
      def body(step, _, c=c):
        r0 = c * _CHUNK + step * _ROWS_PER_STEP
        pairs = ti_v[pl.ds(r0 * 2, 2 * _ROWS_PER_STEP)]
        for r in range(_ROWS_PER_STEP):
          off = pairs[2 * r] * EMBED
          dst = (r0 + r) * EMBED
          for u in range(EMBED // 16):
            rows_v[pl.ds(dst + u * 16, 16)] = table_v[pl.ds(off + u * 16, 16)]
        return 0

      lax.fori_loop(0, _CHUNK // _ROWS_PER_STEP, body, 0, unroll=False)
      out_copies.append(
          pltpu.async_copy(
              rows_v.at[pl.ds(c * _CHUNK * EMBED, _CHUNK * EMBED)],
              out_hbm.at[pl.ds((base + c * _CHUNK) * EMBED, _CHUNK * EMBED)],
              out_sem,
          )
      )
    for cp in out_copies:
      cp.wait()

  return k


_sc_lookup = jax.jit(_make_kernel())


def kernel(time_input, month_table):
  out = _sc_lookup(
      time_input.astype(jnp.int32).reshape(-1), month_table.reshape(-1)
  )
  return out.reshape(BATCH, EMBED)


# speedup vs baseline: 2.3801x; 1.2261x over previous
import functools

import jax
import jax.numpy as jnp
from jax import lax
from jax.experimental import pallas as pl
from jax.experimental.pallas import tpu as pltpu
from jax.experimental.pallas import tpu_sc as plsc

NUM_MONTHS = 12
EMBED = 128
BATCH = 16384

_NC = 2
_NS = 16
_NW = _NC * _NS
_BPW = BATCH // _NW
_CHUNK = 128
_NCHUNK = _BPW // _CHUNK
_ROWS_PER_STEP = 8


def _make_kernel():
  mesh = plsc.VectorSubcoreMesh(core_axis_name="c", subcore_axis_name="s")

  @functools.partial(
      pl.kernel,
      mesh=mesh,
      out_type=jax.ShapeDtypeStruct((BATCH * EMBED,), jnp.float32),
      scratch_types=[
          pltpu.VMEM((NUM_MONTHS * EMBED,), jnp.float32),
          pltpu.VMEM((_BPW * 2,), jnp.int32),
          pltpu.VMEM((_BPW * EMBED,), jnp.float32),
          pltpu.SemaphoreType.DMA,
          pltpu.SemaphoreType.DMA,
      ],
  )
  def k(ti_hbm, table_hbm, out_hbm, table_v, ti_v, rows_v, in_sem, out_sem):
    wid = lax.axis_index("s") * _NC + lax.axis_index("c")
    base = wid * _BPW

    load_table = pltpu.async_copy(table_hbm, table_v, in_sem)
    pltpu.sync_copy(ti_hbm.at[pl.ds(base * 2, _BPW * 2)], ti_v)
    load_table.wait()

    out_copies = []
    for c in range(_NCHUNK):
      @plsc.parallel_loop(c * _CHUNK, (c + 1) * _CHUNK, step=_ROWS_PER_STEP,
                          unroll=2)
      def _(r0):
        pairs = ti_v[pl.ds(r0 * 2, 2 * _ROWS_PER_STEP)]
        for r in range(_ROWS_PER_STEP):
          off = pairs[2 * r] * EMBED
          dst = (r0 + r) * EMBED
          for u in range(EMBED // 16):
            rows_v[pl.ds(dst + u * 16, 16)] = table_v[pl.ds(off + u * 16, 16)]
      out_copies.append(
          pltpu.async_copy(
              rows_v.at[pl.ds(c * _CHUNK * EMBED, _CHUNK * EMBED)],
              out_hbm.at[pl.ds((base + c * _CHUNK) * EMBED, _CHUNK * EMBED)],
              out_sem,
          )
      )
    for cp in out_copies:
      cp.wait()

  return k


_sc_lookup = jax.jit(_make_kernel())


def kernel(time_input, month_table):
  out = _sc_lookup(
      time_input.astype(jnp.int32).reshape(-1), month_table.reshape(-1)
  )
  return out.reshape(BATCH, EMBED)
